# Initial kernel scaffold; baseline (speedup 1.0000x reference)
import jax
import jax.numpy as jnp
from jax.experimental import pallas as pl


def kernel(hiddens, keys, encoded_sents, indices, U, V, W):
    raise NotImplementedError("write your pallas kernel here")


# speedup vs baseline: 2.2419x; 2.2419x over previous
import functools

import jax
import jax.numpy as jnp
from jax import lax
from jax.experimental import pallas as pl
from jax.experimental.pallas import tpu as pltpu
from jax.experimental.pallas import tpu_sc as plsc

M = 16384
E = 20
D = 64
P = 4096
ED = E * D

_NW = 32
_RPW = P // _NW
_CH = 64
_BM = 128
_NB = M // _BM
_PB = 512


def _gather_body(h_hbm, k_hbm, idx_hbm, out_h, out_k, idx_v, buf, sem):
    wid = lax.axis_index("s") * 2 + lax.axis_index("c")
    base = wid * _RPW
    for c in range(_RPW // _CH):
        off = base + c * _CH
        pltpu.sync_copy(idx_hbm.at[pl.ds(off, _CH)], idx_v)
        pltpu.async_copy(h_hbm.at[idx_v], buf, sem).wait()
        pltpu.sync_copy(buf, out_h.at[pl.ds(off, _CH)])
        pltpu.async_copy(k_hbm.at[idx_v], buf, sem).wait()
        pltpu.sync_copy(buf, out_k.at[pl.ds(off, _CH)])


@functools.cache
def _sc_gather():
    return pl.kernel(
        _gather_body,
        mesh=plsc.VectorSubcoreMesh(core_axis_name="c", subcore_axis_name="s"),
        out_type=[
            jax.ShapeDtypeStruct((P, ED), jnp.float32),
            jax.ShapeDtypeStruct((P, ED), jnp.float32),
        ],
        scratch_types=[
            pltpu.VMEM((_CH,), jnp.int32),
            pltpu.VMEM((_CH, ED), jnp.float32),
            pltpu.SemaphoreType.DMA,
        ],
    )


def _prep_body(es_ref, u_ref, v_ref, w_ref, esw_ref, a_ref):
    esw_ref[...] = jnp.dot(es_ref[...], w_ref[...],
                           preferred_element_type=jnp.float32)
    a_ref[...] = u_ref[...] + v_ref[...]


_prep = pl.pallas_call(
    _prep_body,
    out_shape=[
        jax.ShapeDtypeStruct((P, D), jnp.float32),
        jax.ShapeDtypeStruct((D, D), jnp.float32),
    ],
)


def _htilda_body(chf_ref, esw_ref, a_ref, out_ref):
    out_ref[...] = jnp.maximum(
        jnp.dot(chf_ref[...], a_ref[...], preferred_element_type=jnp.float32)
        + esw_ref[...], 0.0)


_htilda = pl.pallas_call(
    _htilda_body,
    grid=(E,),
    in_specs=[
        pl.BlockSpec((P, D), lambda k: (k, 0)),
        pl.BlockSpec((P, D), lambda k: (0, 0)),
        pl.BlockSpec((D, D), lambda k: (0, 0)),
    ],
    out_specs=pl.BlockSpec((P, D), lambda k: (k, 0)),
    out_shape=jax.ShapeDtypeStruct((P * E, D), jnp.float32),
)


def _gate_body(ch_ref, ck_ref, es_ref, h_ref, r_ref, s_ref, st_ref, out_ref):
    es_rep = jnp.dot(es_ref[...], r_ref[...],
                     preferred_element_type=jnp.float32)
    pre = ((ch_ref[...] + ck_ref[...]) * es_rep)
    gates = jax.nn.sigmoid(jnp.dot(pre, s_ref[...],
                                   preferred_element_type=jnp.float32))
    out_ref[...] = jnp.dot(gates, st_ref[...],
                           preferred_element_type=jnp.float32) * h_ref[...]


_gate = pl.pallas_call(
    _gate_body,
    grid=(P // _PB,),
    in_specs=[
        pl.BlockSpec((_PB, ED), lambda k: (k, 0)),
        pl.BlockSpec((_PB, ED), lambda k: (k, 0)),
        pl.BlockSpec((_PB, D), lambda k: (k, 0)),
        pl.BlockSpec((_PB, ED), lambda k: (k, 0)),
        pl.BlockSpec((D, ED), lambda k: (0, 0)),
        pl.BlockSpec((ED, E), lambda k: (0, 0)),
        pl.BlockSpec((E, ED), lambda k: (0, 0)),
    ],
    out_specs=pl.BlockSpec((_PB, ED), lambda k: (k, 0)),
    out_shape=jax.ShapeDtypeStruct((P, ED), jnp.float32),
)


def _apply_body(sidx_ref, order_ref, starts_ref, hid_ref, upd_ref,
                s_ref, st_ref, out_ref):
    b = pl.program_id(0)
    out_ref[...] = hid_ref[...]
    base = b * _BM

    def add_one(t, carry):
        r = sidx_ref[t] - base
        j = order_ref[t]
        out_ref[pl.ds(r, 1), :] += upd_ref[pl.ds(j, 1), :]
        return carry

    lax.fori_loop(starts_ref[b], starts_ref[b + 1], add_one, 0)

    x = out_ref[...]
    ss = jnp.dot(x * x, s_ref[...], preferred_element_type=jnp.float32)
    scale = lax.rsqrt(jnp.maximum(ss, 1e-12))
    out_ref[...] = x * jnp.dot(scale, st_ref[...],
                               preferred_element_type=jnp.float32)


_apply = pl.pallas_call(
    _apply_body,
    grid_spec=pltpu.PrefetchScalarGridSpec(
        num_scalar_prefetch=3,
        grid=(_NB,),
        in_specs=[
            pl.BlockSpec((_BM, ED), lambda b, *_: (b, 0)),
            pl.BlockSpec((P, ED), lambda b, *_: (0, 0)),
            pl.BlockSpec((ED, E), lambda b, *_: (0, 0)),
            pl.BlockSpec((E, ED), lambda b, *_: (0, 0)),
        ],
        out_specs=pl.BlockSpec((_BM, ED), lambda b, *_: (b, 0)),
    ),
    out_shape=jax.ShapeDtypeStruct((M, ED), jnp.float32),
)


def kernel(hiddens, keys, encoded_sents, indices, U, V, W):
    idx = indices.astype(jnp.int32)
    h2 = hiddens.reshape(M, ED)
    k2 = keys.reshape(M, ED)

    order = jnp.argsort(idx).astype(jnp.int32)
    sidx = jnp.take(idx, order)
    edges = jnp.arange(0, M + _BM, _BM, dtype=jnp.int32)
    starts = jnp.searchsorted(sidx, edges).astype(jnp.int32)

    eyeD = jnp.eye(D, dtype=jnp.float32)
    R = jnp.tile(eyeD, (1, E))
    S = jnp.kron(jnp.eye(E, dtype=jnp.float32),
                 jnp.ones((D, 1), jnp.float32))
    ST = S.T

    ch, ck = _sc_gather()(h2, k2, idx)
    esw, A = _prep(encoded_sents, U, V, W)
    hflat = _htilda(ch.reshape(P * E, D), esw, A)
    upd = _gate(ch, ck, encoded_sents, hflat.reshape(P, ED), R, S, ST)
    out = _apply(sidx, order, starts, h2, upd, S, ST)
    return out.reshape(M, E, D)
